# baseline (speedup 1.0000x reference)
import jax
import jax.numpy as jnp
from jax import lax
from jax.experimental import pallas as pl
from jax.experimental.pallas import tpu as pltpu
from jax.experimental.pallas import tpu_sc as plsc

LANES = 16
NUM_WORKERS = 32
SEQ = 512
SEQS_PER_WORKER = 4
CHUNK = 16
NCHUNK = (SEQS_PER_WORKER * SEQ) // CHUNK
CHUNKS_PER_SEQ = SEQ // CHUNK
D = 1024


def _sc_pool_body(emb_hbm, ids_hbm, out_hbm,
                  idx_v, buf0, buf1, buf2, buf3, acc, sem0, sem1, sem2, sem3):
    wid = lax.axis_index("s") * 2 + lax.axis_index("c")
    base = wid * (SEQS_PER_WORKER * SEQ)
    pltpu.sync_copy(ids_hbm.at[pl.ds(base, SEQS_PER_WORKER * SEQ)], idx_v)

    def gather_start(chunk, buf, sem):
        idx_sl = idx_v.at[pl.ds(chunk * CHUNK, CHUNK)]
        pltpu.make_async_copy(emb_hbm.at[idx_sl], buf, sem).start()

    def gather_wait(chunk, buf, sem):
        idx_sl = idx_v.at[pl.ds(chunk * CHUNK, CHUNK)]
        pltpu.make_async_copy(emb_hbm.at[idx_sl], buf, sem).wait()

    def accum(buf):
        @plsc.parallel_loop(0, D // LANES, unroll=2)
        def _(g):
            sl = pl.ds(g * LANES, LANES)
            vals = [buf[r, sl] for r in range(CHUNK)]
            while len(vals) > 1:
                vals = [a + b for a, b in zip(vals[::2], vals[1::2])]
            plsc.addupdate(acc.at[0, sl], vals[0])

    def zero_acc():
        @pl.loop(0, D // LANES)
        def _(g):
            sl = pl.ds(g * LANES, LANES)
            acc[0, sl] = jnp.zeros((LANES,), jnp.float32)

    zero_acc()

    ring = ((buf0, sem0), (buf1, sem1), (buf2, sem2), (buf3, sem3))
    for k, (buf, sem) in enumerate(ring):
        gather_start(k, buf, sem)

    @pl.loop(0, NCHUNK, step=len(ring))
    def _(c):
        for k, (buf, sem) in enumerate(ring):
            gather_wait(c + k, buf, sem)

            @pl.when(c + k + len(ring) < NCHUNK)
            def _():
                gather_start(c + k + len(ring), buf, sem)

            @pl.when((c + k) % CHUNKS_PER_SEQ == CHUNKS_PER_SEQ - 1)
            def _():
                row = wid * SEQS_PER_WORKER + (c + k) // CHUNKS_PER_SEQ
                pltpu.sync_copy(acc, out_hbm.at[pl.ds(row, 1)])
                zero_acc()


def _sc_pool(emb, ids_flat):
    n_rows = ids_flat.shape[0] // SEQ
    run = pl.kernel(
        _sc_pool_body,
        out_type=jax.ShapeDtypeStruct((n_rows, D), jnp.float32),
        mesh=plsc.VectorSubcoreMesh(core_axis_name="c", subcore_axis_name="s"),
        scratch_types=[
            pltpu.VMEM((SEQS_PER_WORKER * SEQ,), jnp.int32),
            pltpu.VMEM((CHUNK, D), jnp.float32),
            pltpu.VMEM((CHUNK, D), jnp.float32),
            pltpu.VMEM((CHUNK, D), jnp.float32),
            pltpu.VMEM((CHUNK, D), jnp.float32),
            pltpu.VMEM((1, D), jnp.float32),
            pltpu.SemaphoreType.DMA,
            pltpu.SemaphoreType.DMA,
            pltpu.SemaphoreType.DMA,
            pltpu.SemaphoreType.DMA,
        ],
    )
    return run(emb, ids_flat)


def _mlp_body(x_ref, w1_ref, b1_ref, w2_ref, b2_ref, o_ref):
    x = x_ref[...] * (1.0 / SEQ)
    h = jnp.dot(x, w1_ref[...], preferred_element_type=jnp.float32) + b1_ref[...]
    h = jnp.maximum(h, 0.0)
    o_ref[...] = jnp.dot(h, w2_ref[...], preferred_element_type=jnp.float32) + b2_ref[...]


def _mlp(pooled, W1, b1, W2, b2):
    n = pooled.shape[0]
    return pl.pallas_call(
        _mlp_body,
        out_shape=jax.ShapeDtypeStruct((n, 1), jnp.float32),
    )(pooled, W1, b1.reshape(1, -1), W2, b2.reshape(1, 1))


def kernel(input_ids, emb, W1, b1, W2, b2):
    batch, choices, seq = input_ids.shape
    ids_flat = input_ids.reshape(batch * choices * seq).astype(jnp.int32)
    pooled = _sc_pool(emb, ids_flat)
    logits = _mlp(pooled, W1, b1, W2, b2)
    return logits.reshape(batch, choices)


# speedup vs baseline: 1.0491x; 1.0491x over previous
import jax
import jax.numpy as jnp
from jax import lax
from jax.experimental import pallas as pl
from jax.experimental.pallas import tpu as pltpu
from jax.experimental.pallas import tpu_sc as plsc

LANES = 16
NUM_WORKERS = 32
SEQ = 512
SEQS_PER_WORKER = 4
CHUNK = 8
NCHUNK = (SEQS_PER_WORKER * SEQ) // CHUNK
CHUNKS_PER_SEQ = SEQ // CHUNK
D = 1024


def _sc_pool_body(emb_hbm, ids_hbm, out_hbm,
                  idx_v, buf0, buf1, buf2, buf3, buf4, buf5, buf6, buf7, acc,
                  sem0, sem1, sem2, sem3, sem4, sem5, sem6, sem7):
    wid = lax.axis_index("s") * 2 + lax.axis_index("c")
    base = wid * (SEQS_PER_WORKER * SEQ)
    pltpu.sync_copy(ids_hbm.at[pl.ds(base, SEQS_PER_WORKER * SEQ)], idx_v)

    def gather_start(chunk, buf, sem):
        idx_sl = idx_v.at[pl.ds(chunk * CHUNK, CHUNK)]
        pltpu.make_async_copy(emb_hbm.at[idx_sl], buf, sem).start()

    def gather_wait(chunk, buf, sem):
        idx_sl = idx_v.at[pl.ds(chunk * CHUNK, CHUNK)]
        pltpu.make_async_copy(emb_hbm.at[idx_sl], buf, sem).wait()

    def accum(buf):
        @plsc.parallel_loop(0, D // LANES, unroll=2)
        def _(g):
            sl = pl.ds(g * LANES, LANES)
            vals = [buf[r, sl] for r in range(CHUNK)]
            while len(vals) > 1:
                vals = [a + b for a, b in zip(vals[::2], vals[1::2])]
            plsc.addupdate(acc.at[0, sl], vals[0])

    def zero_acc():
        @pl.loop(0, D // LANES)
        def _(g):
            sl = pl.ds(g * LANES, LANES)
            acc[0, sl] = jnp.zeros((LANES,), jnp.float32)

    zero_acc()

    ring = ((buf0, sem0), (buf1, sem1), (buf2, sem2), (buf3, sem3),
            (buf4, sem4), (buf5, sem5), (buf6, sem6), (buf7, sem7))
    for k, (buf, sem) in enumerate(ring):
        gather_start(k, buf, sem)

    @pl.loop(0, NCHUNK, step=len(ring))
    def _(c):
        for k, (buf, sem) in enumerate(ring):
            gather_wait(c + k, buf, sem)
            accum(buf)

            @pl.when(c + k + len(ring) < NCHUNK)
            def _():
                gather_start(c + k + len(ring), buf, sem)

            @pl.when((c + k) % CHUNKS_PER_SEQ == CHUNKS_PER_SEQ - 1)
            def _():
                row = wid * SEQS_PER_WORKER + (c + k) // CHUNKS_PER_SEQ
                pltpu.sync_copy(acc, out_hbm.at[pl.ds(row, 1)])
                zero_acc()


def _sc_pool(emb, ids_flat):
    n_rows = ids_flat.shape[0] // SEQ
    run = pl.kernel(
        _sc_pool_body,
        out_type=jax.ShapeDtypeStruct((n_rows, D), jnp.float32),
        mesh=plsc.VectorSubcoreMesh(core_axis_name="c", subcore_axis_name="s"),
        scratch_types=[
            pltpu.VMEM((SEQS_PER_WORKER * SEQ,), jnp.int32),
            pltpu.VMEM((CHUNK, D), jnp.float32),
            pltpu.VMEM((CHUNK, D), jnp.float32),
            pltpu.VMEM((CHUNK, D), jnp.float32),
            pltpu.VMEM((CHUNK, D), jnp.float32),
            pltpu.VMEM((CHUNK, D), jnp.float32),
            pltpu.VMEM((CHUNK, D), jnp.float32),
            pltpu.VMEM((CHUNK, D), jnp.float32),
            pltpu.VMEM((CHUNK, D), jnp.float32),
            pltpu.VMEM((1, D), jnp.float32),
            pltpu.SemaphoreType.DMA,
            pltpu.SemaphoreType.DMA,
            pltpu.SemaphoreType.DMA,
            pltpu.SemaphoreType.DMA,
            pltpu.SemaphoreType.DMA,
            pltpu.SemaphoreType.DMA,
            pltpu.SemaphoreType.DMA,
            pltpu.SemaphoreType.DMA,
        ],
    )
    return run(emb, ids_flat)


def _mlp_body(x_ref, w1_ref, b1_ref, w2_ref, b2_ref, o_ref):
    x = x_ref[...] * (1.0 / SEQ)
    h = jnp.dot(x, w1_ref[...], preferred_element_type=jnp.float32) + b1_ref[...]
    h = jnp.maximum(h, 0.0)
    o_ref[...] = jnp.dot(h, w2_ref[...], preferred_element_type=jnp.float32) + b2_ref[...]


def _mlp(pooled, W1, b1, W2, b2):
    n = pooled.shape[0]
    return pl.pallas_call(
        _mlp_body,
        out_shape=jax.ShapeDtypeStruct((n, 1), jnp.float32),
    )(pooled, W1, b1.reshape(1, -1), W2, b2.reshape(1, 1))


def kernel(input_ids, emb, W1, b1, W2, b2):
    batch, choices, seq = input_ids.shape
    ids_flat = input_ids.reshape(batch * choices * seq).astype(jnp.int32)
    pooled = _sc_pool(emb, ids_flat)
    logits = _mlp(pooled, W1, b1, W2, b2)
    return logits.reshape(batch, choices)


# speedup vs baseline: 1.0708x; 1.0207x over previous
import jax
import jax.numpy as jnp
from jax import lax
from jax.experimental import pallas as pl
from jax.experimental.pallas import tpu as pltpu
from jax.experimental.pallas import tpu_sc as plsc

LANES = 16
NUM_WORKERS = 32
SEQ = 512
SEQS_PER_WORKER = 4
CHUNK = 16
NCHUNK = (SEQS_PER_WORKER * SEQ) // CHUNK
CHUNKS_PER_SEQ = SEQ // CHUNK
D = 1024


def _sc_pool_body(emb_hbm, ids_hbm, out_hbm,
                  idx_v, buf0, buf1, buf2, buf3, acc, sem0, sem1, sem2, sem3):
    wid = lax.axis_index("s") * 2 + lax.axis_index("c")
    base = wid * (SEQS_PER_WORKER * SEQ)
    pltpu.sync_copy(ids_hbm.at[pl.ds(base, SEQS_PER_WORKER * SEQ)], idx_v)

    def gather_start(chunk, buf, sem):
        idx_sl = idx_v.at[pl.ds(chunk * CHUNK, CHUNK)]
        pltpu.make_async_copy(emb_hbm.at[idx_sl], buf, sem).start()

    def gather_wait(chunk, buf, sem):
        idx_sl = idx_v.at[pl.ds(chunk * CHUNK, CHUNK)]
        pltpu.make_async_copy(emb_hbm.at[idx_sl], buf, sem).wait()

    def accum(buf):
        @plsc.parallel_loop(0, D // LANES, unroll=2)
        def _(g):
            sl = pl.ds(g * LANES, LANES)
            vals = [buf[r, sl] for r in range(CHUNK)]
            while len(vals) > 1:
                vals = [a + b for a, b in zip(vals[::2], vals[1::2])]
            plsc.addupdate(acc.at[0, sl], vals[0])

    def zero_acc():
        @pl.loop(0, D // LANES)
        def _(g):
            sl = pl.ds(g * LANES, LANES)
            acc[0, sl] = jnp.zeros((LANES,), jnp.float32)

    zero_acc()

    ring = ((buf0, sem0), (buf1, sem1), (buf2, sem2), (buf3, sem3))
    for k, (buf, sem) in enumerate(ring):
        gather_start(k, buf, sem)

    @pl.loop(0, NCHUNK, step=len(ring))
    def _(c):
        for k, (buf, sem) in enumerate(ring):
            gather_wait(c + k, buf, sem)
            accum(buf)

            @pl.when(c + k + len(ring) < NCHUNK)
            def _():
                gather_start(c + k + len(ring), buf, sem)

            @pl.when((c + k) % CHUNKS_PER_SEQ == CHUNKS_PER_SEQ - 1)
            def _():
                row = wid * SEQS_PER_WORKER + (c + k) // CHUNKS_PER_SEQ
                pltpu.sync_copy(acc, out_hbm.at[pl.ds(row, 1)])
                zero_acc()


def _sc_pool(emb, ids_flat):
    n_rows = ids_flat.shape[0] // SEQ
    run = pl.kernel(
        _sc_pool_body,
        out_type=jax.ShapeDtypeStruct((n_rows, D), jnp.float32),
        mesh=plsc.VectorSubcoreMesh(core_axis_name="c", subcore_axis_name="s"),
        scratch_types=[
            pltpu.VMEM((SEQS_PER_WORKER * SEQ,), jnp.int32),
            pltpu.VMEM((CHUNK, D), jnp.float32),
            pltpu.VMEM((CHUNK, D), jnp.float32),
            pltpu.VMEM((CHUNK, D), jnp.float32),
            pltpu.VMEM((CHUNK, D), jnp.float32),
            pltpu.VMEM((1, D), jnp.float32),
            pltpu.SemaphoreType.DMA,
            pltpu.SemaphoreType.DMA,
            pltpu.SemaphoreType.DMA,
            pltpu.SemaphoreType.DMA,
        ],
    )
    return run(emb, ids_flat)


def _mlp_body(x_ref, w1_ref, b1_ref, w2_ref, b2_ref, o_ref):
    x = x_ref[...] * (1.0 / SEQ)
    h = jnp.dot(x, w1_ref[...], preferred_element_type=jnp.float32) + b1_ref[...]
    h = jnp.maximum(h, 0.0)
    o_ref[...] = jnp.dot(h, w2_ref[...], preferred_element_type=jnp.float32) + b2_ref[...]


def _mlp(pooled, W1, b1, W2, b2):
    n = pooled.shape[0]
    return pl.pallas_call(
        _mlp_body,
        out_shape=jax.ShapeDtypeStruct((n, 1), jnp.float32),
    )(pooled, W1, b1.reshape(1, -1), W2, b2.reshape(1, 1))


def kernel(input_ids, emb, W1, b1, W2, b2):
    batch, choices, seq = input_ids.shape
    ids_flat = input_ids.reshape(batch * choices * seq).astype(jnp.int32)
    pooled = _sc_pool(emb, ids_flat)
    logits = _mlp(pooled, W1, b1, W2, b2)
    return logits.reshape(batch, choices)


# speedup vs baseline: 1.0767x; 1.0055x over previous
import jax
import jax.numpy as jnp
from jax import lax
from jax.experimental import pallas as pl
from jax.experimental.pallas import tpu as pltpu
from jax.experimental.pallas import tpu_sc as plsc

LANES = 16
NUM_WORKERS = 32
SEQ = 512
SEQS_PER_WORKER = 4
CHUNK = 16
NCHUNK = (SEQS_PER_WORKER * SEQ) // CHUNK
CHUNKS_PER_SEQ = SEQ // CHUNK
D = 1024


def _sc_pool_body(emb_hbm, ids_hbm, out_hbm,
                  idx_v, buf0, buf1, buf2, buf3, acc, sem0, sem1, sem2, sem3):
    wid = lax.axis_index("s") * 2 + lax.axis_index("c")
    base = wid * (SEQS_PER_WORKER * SEQ)
    pltpu.sync_copy(ids_hbm.at[pl.ds(base, SEQS_PER_WORKER * SEQ)], idx_v)

    def gather_start(chunk, buf, sem):
        idx_sl = idx_v.at[pl.ds(chunk * CHUNK, CHUNK)]
        pltpu.make_async_copy(emb_hbm.at[idx_sl], buf, sem).start()

    def gather_wait(chunk, buf, sem):
        idx_sl = idx_v.at[pl.ds(chunk * CHUNK, CHUNK)]
        pltpu.make_async_copy(emb_hbm.at[idx_sl], buf, sem).wait()

    def accum(buf):
        @plsc.parallel_loop(0, D // LANES, unroll=1)
        def _(g):
            sl = pl.ds(g * LANES, LANES)
            vals = [buf[r, sl] for r in range(CHUNK)]
            while len(vals) > 1:
                vals = [a + b for a, b in zip(vals[::2], vals[1::2])]
            plsc.addupdate(acc.at[0, sl], vals[0])

    def zero_acc():
        @pl.loop(0, D // LANES)
        def _(g):
            sl = pl.ds(g * LANES, LANES)
            acc[0, sl] = jnp.zeros((LANES,), jnp.float32)

    zero_acc()

    ring = ((buf0, sem0), (buf1, sem1), (buf2, sem2), (buf3, sem3))
    for k, (buf, sem) in enumerate(ring):
        gather_start(k, buf, sem)

    @pl.loop(0, NCHUNK, step=len(ring))
    def _(c):
        for k, (buf, sem) in enumerate(ring):
            gather_wait(c + k, buf, sem)
            accum(buf)

            @pl.when(c + k + len(ring) < NCHUNK)
            def _():
                gather_start(c + k + len(ring), buf, sem)

            @pl.when((c + k) % CHUNKS_PER_SEQ == CHUNKS_PER_SEQ - 1)
            def _():
                row = wid * SEQS_PER_WORKER + (c + k) // CHUNKS_PER_SEQ
                pltpu.sync_copy(acc, out_hbm.at[pl.ds(row, 1)])
                zero_acc()


def _sc_pool(emb, ids_flat):
    n_rows = ids_flat.shape[0] // SEQ
    run = pl.kernel(
        _sc_pool_body,
        out_type=jax.ShapeDtypeStruct((n_rows, D), jnp.float32),
        mesh=plsc.VectorSubcoreMesh(core_axis_name="c", subcore_axis_name="s"),
        scratch_types=[
            pltpu.VMEM((SEQS_PER_WORKER * SEQ,), jnp.int32),
            pltpu.VMEM((CHUNK, D), jnp.float32),
            pltpu.VMEM((CHUNK, D), jnp.float32),
            pltpu.VMEM((CHUNK, D), jnp.float32),
            pltpu.VMEM((CHUNK, D), jnp.float32),
            pltpu.VMEM((1, D), jnp.float32),
            pltpu.SemaphoreType.DMA,
            pltpu.SemaphoreType.DMA,
            pltpu.SemaphoreType.DMA,
            pltpu.SemaphoreType.DMA,
        ],
    )
    return run(emb, ids_flat)


def _mlp_body(x_ref, w1_ref, b1_ref, w2_ref, b2_ref, o_ref):
    x = x_ref[...] * (1.0 / SEQ)
    h = jnp.dot(x, w1_ref[...], preferred_element_type=jnp.float32) + b1_ref[...]
    h = jnp.maximum(h, 0.0)
    o_ref[...] = jnp.dot(h, w2_ref[...], preferred_element_type=jnp.float32) + b2_ref[...]


def _mlp(pooled, W1, b1, W2, b2):
    n = pooled.shape[0]
    return pl.pallas_call(
        _mlp_body,
        out_shape=jax.ShapeDtypeStruct((n, 1), jnp.float32),
    )(pooled, W1, b1.reshape(1, -1), W2, b2.reshape(1, 1))


def kernel(input_ids, emb, W1, b1, W2, b2):
    batch, choices, seq = input_ids.shape
    ids_flat = input_ids.reshape(batch * choices * seq).astype(jnp.int32)
    pooled = _sc_pool(emb, ids_flat)
    logits = _mlp(pooled, W1, b1, W2, b2)
    return logits.reshape(batch, choices)
